# baseline (speedup 1.0000x reference)
import functools
import math

import jax
import jax.numpy as jnp
from jax import lax
from jax.experimental import pallas as pl
from jax.experimental.pallas import tpu as pltpu
from jax.experimental.pallas import tpu_sc as plsc

D_MODEL = 64
SCALE = math.sqrt(D_MODEL)

NC = 2
NS = 16
NW = NC * NS
LANES = 16

V = 1000000
VT_FULL = V // 128
VT_TAIL = V - VT_FULL * 128

BLK = 128
NBUF = 4
K1_NBUF = 2


def _make_relayout():
  mesh = plsc.VectorSubcoreMesh(core_axis_name="c", subcore_axis_name="s")
  base_n = VT_FULL // NW
  extra = VT_FULL - base_n * NW

  @functools.partial(
      pl.kernel,
      mesh=mesh,
      out_type=jax.ShapeDtypeStruct((V, 128), jnp.float32),
      scratch_types=[
          pltpu.VMEM((K1_NBUF, D_MODEL, 128), jnp.float32),
          pltpu.VMEM((K1_NBUF, 128, 129), jnp.float32),
          [pltpu.SemaphoreType.DMA] * K1_NBUF,
          [pltpu.SemaphoreType.DMA] * K1_NBUF,
      ],
      compiler_params=pltpu.CompilerParams(use_tc_tiling_on_sc=True,
                                           needs_layout_passes=False),
  )
  def k1(tabT_hbm, tail_hbm, out_hbm, in_v, tr_v, isems, osems):
    wid = lax.axis_index("s") * NC + lax.axis_index("c")
    n_mine = base_n + jnp.where(wid < extra, 1, 0)
    t0 = wid * base_n + jnp.minimum(wid, extra)

    lane = lax.iota(jnp.int32, LANES)
    vls = [lane + g * LANES for g in range(128 // LANES)]

    def fire_in(t, b):
      pltpu.make_async_copy(
          tabT_hbm.at[:, pl.ds(t * 128, 128)], in_v.at[b], isems[b]).start()

    def wait_in(b):
      pltpu.make_async_copy(
          tabT_hbm.at[:, pl.ds(0, 128)], in_v.at[b], isems[b]).wait()

    def fire_out(t, b):
      pltpu.make_async_copy(
          tr_v.at[b].at[pl.ds(0, 128), pl.ds(0, 128)],
          out_hbm.at[pl.ds(t * 128, 128)], osems[b]).start()

    def wait_out(b):
      pltpu.make_async_copy(
          tr_v.at[b].at[pl.ds(0, 128), pl.ds(0, 128)],
          out_hbm.at[pl.ds(0, 128)], osems[b]).wait()

    def transpose_block(b, d_lo, d_hi):
      @plsc.parallel_loop(d_lo, d_hi)
      def dbody(d):
        cold = jnp.full((LANES,), d, jnp.int32)
        for g in range(128 // LANES):
          vec = in_v[b, d, pl.ds(g * LANES, LANES)]
          plsc.store_scatter(tr_v.at[b], [vls[g], cold], vec)

    fire_in(t0, 0)

    def body(n, b, nb):
      t = t0 + n
      wait_in(b)

      @pl.when(n + 1 < n_mine)
      def _():
        @pl.when(n >= K1_NBUF - 1)
        def _():
          wait_out(nb)
        fire_in(t + 1, nb)

      transpose_block(b, 0, D_MODEL)
      fire_out(t, b)

    def outer(g0, carry):
      for b in range(K1_NBUF):
        body(g0 * K1_NBUF + b, b, (b + 1) % K1_NBUF)
      return carry

    lax.fori_loop(0, base_n // K1_NBUF, outer, 0)

    @pl.when(n_mine > base_n)
    def _():
      body(base_n, 0, 1)

    for b in range(K1_NBUF):
      wait_out(b)

    @pl.when(wid == NW - 1)
    def _():
      pltpu.make_async_copy(tail_hbm, in_v.at[0], isems[0]).start()
      pltpu.make_async_copy(tail_hbm, in_v.at[0], isems[0]).wait()
      transpose_block(0, 0, D_MODEL)
      pltpu.make_async_copy(
          tr_v.at[0].at[pl.ds(64, 64), pl.ds(0, 128)],
          out_hbm.at[pl.ds(VT_FULL * 128, 64)], osems[0]).start()
      pltpu.make_async_copy(
          tr_v.at[0].at[pl.ds(0, 64), pl.ds(0, 128)],
          out_hbm.at[pl.ds(0, 64)], osems[0]).wait()

  return k1


def _make_gather(B: int, S: int):
  bi_blocks = B // BLK
  bi_per_w = bi_blocks // NW
  n_chunks = S * bi_per_w
  idx_per_w = S * bi_per_w * BLK
  mesh = plsc.VectorSubcoreMesh(core_axis_name="c", subcore_axis_name="s")

  @functools.partial(
      pl.kernel,
      mesh=mesh,
      out_type=jax.ShapeDtypeStruct(
          (S, D_MODEL // 8, bi_blocks, 8, BLK), jnp.float32),
      scratch_types=[
          pltpu.VMEM((idx_per_w,), jnp.int32),
          pltpu.VMEM((NBUF, BLK, 128), jnp.float32),
          pltpu.VMEM((NBUF, D_MODEL, BLK + 1), jnp.float32),
          [pltpu.SemaphoreType.DMA] * NBUF,
          [pltpu.SemaphoreType.DMA] * NBUF,
          pltpu.SemaphoreType.DMA,
      ],
      compiler_params=pltpu.CompilerParams(use_tc_tiling_on_sc=False,
                                           needs_layout_passes=False),
  )
  def k2(idx_hbm, table_hbm, out_hbm, idx_v, rows_v, tbuf_v,
         gsems, osems, isem):
    wid = lax.axis_index("s") * NC + lax.axis_index("c")

    icopies = []
    for j in range(S):
      cp = pltpu.make_async_copy(
          idx_hbm.at[pl.ds(j * B + wid * (bi_per_w * BLK), bi_per_w * BLK)],
          idx_v.at[pl.ds(j * (bi_per_w * BLK), bi_per_w * BLK)],
          isem,
      )
      cp.start()
      icopies.append(cp)
    for cp in icopies:
      cp.wait()

    lane = lax.iota(jnp.int32, LANES)
    dcols = [lane + db * LANES for db in range(D_MODEL // LANES)]

    def fire_gather(k, b):
      pltpu.make_async_copy(
          table_hbm.at[idx_v.at[pl.ds(k * BLK, BLK)]],
          rows_v.at[b], gsems[b]).start()

    def wait_gather(b):
      pltpu.make_async_copy(
          table_hbm.at[idx_v.at[pl.ds(0, BLK)]],
          rows_v.at[b], gsems[b]).wait()

    def fire_scatter(k, b):
      j = k >> 2
      bi = wid * bi_per_w + (k & 3)
      for bd in range(D_MODEL // 8):
        pltpu.make_async_copy(
            tbuf_v.at[b].at[pl.ds(bd * 8, 8), pl.ds(0, BLK)],
            out_hbm.at[j, bd, bi], osems[b]).start()

    def wait_scatter(b):
      for bd in range(D_MODEL // 8):
        pltpu.make_async_copy(
            tbuf_v.at[b].at[pl.ds(bd * 8, 8), pl.ds(0, BLK)],
            out_hbm.at[0, bd, 0], osems[b]).wait()

    fire_gather(0, 0)

    def chunk(k, b, nb):
      wait_gather(b)

      @pl.when(k + 1 < n_chunks)
      def _():
        @pl.when(k >= NBUF - 1)
        def _():
          wait_scatter(nb)
        fire_gather(k + 1, nb)

      @plsc.parallel_loop(0, BLK)
      def rbody(i):
        coli = jnp.full((LANES,), i, jnp.int32)
        for db in range(D_MODEL // LANES):
          vec = rows_v[b, i, pl.ds(db * LANES, LANES)]
          plsc.store_scatter(tbuf_v.at[b], [dcols[db], coli], vec * SCALE)

      fire_scatter(k, b)

    def outer(g0, carry):
      for b in range(NBUF):
        chunk(g0 * NBUF + b, b, (b + 1) % NBUF)
      return carry

    lax.fori_loop(0, n_chunks // NBUF, outer, 0)
    for b in range(NBUF):
      wait_scatter(b)

  return k2


def kernel(x, table):
  B, S = x.shape
  idxT = x.T.reshape(-1).astype(jnp.int32)
  tail = table[V - 128:].T
  padded = _make_relayout()(table.T, tail)
  out5 = _make_gather(B, S)(idxT, padded)
  return out5.transpose(2, 4, 0, 1, 3).reshape(B, S, D_MODEL)


# speedup vs baseline: 1.5946x; 1.5946x over previous
import functools
import math

import jax
import jax.numpy as jnp
from jax import lax
from jax.experimental import pallas as pl
from jax.experimental.pallas import tpu as pltpu
from jax.experimental.pallas import tpu_sc as plsc

D_MODEL = 64
SCALE = math.sqrt(D_MODEL)

NC = 2
NS = 16
NW = NC * NS
LANES = 16

BLK = 128
NBUF = 4


def _make_kernel(B: int, S: int):
  bi_blocks = B // BLK
  bi_per_w = bi_blocks // NW
  n_chunks = S * bi_per_w
  idx_per_w = S * bi_per_w * BLK
  mesh = plsc.VectorSubcoreMesh(core_axis_name="c", subcore_axis_name="s")

  @functools.partial(
      pl.kernel,
      mesh=mesh,
      out_type=jax.ShapeDtypeStruct(
          (S, D_MODEL // 8, bi_blocks, 8, BLK), jnp.float32),
      scratch_types=[
          pltpu.VMEM((idx_per_w,), jnp.int32),
          pltpu.VMEM((NBUF, BLK, D_MODEL), jnp.float32),
          pltpu.VMEM((NBUF, D_MODEL, BLK + 1), jnp.float32),
          [pltpu.SemaphoreType.DMA] * NBUF,
          [pltpu.SemaphoreType.DMA] * NBUF,
          pltpu.SemaphoreType.DMA,
      ],
      compiler_params=pltpu.CompilerParams(use_tc_tiling_on_sc=False, needs_layout_passes=False),
  )
  def kern(idx_hbm, table_hbm, out_hbm, idx_v, rows_v, tbuf_v,
           gsems, osems, isem):
    wid = lax.axis_index("s") * NC + lax.axis_index("c")

    icopies = []
    for j in range(S):
      cp = pltpu.make_async_copy(
          idx_hbm.at[pl.ds(j * B + wid * (bi_per_w * BLK), bi_per_w * BLK)],
          idx_v.at[pl.ds(j * (bi_per_w * BLK), bi_per_w * BLK)],
          isem,
      )
      cp.start()
      icopies.append(cp)
    for cp in icopies:
      cp.wait()

    lane = lax.iota(jnp.int32, LANES)
    perms = [(lane + c) & (LANES - 1) for c in range(LANES)]
    dcols = [lane + db * LANES for db in range(D_MODEL // LANES)]

    def fire_gather(k, b):
      pltpu.make_async_copy(
          table_hbm.at[idx_v.at[pl.ds(k * BLK, BLK)]],
          rows_v.at[b],
          gsems[b],
      ).start()

    def wait_gather(b):
      pltpu.make_async_copy(
          table_hbm.at[idx_v.at[pl.ds(0, BLK)]],
          rows_v.at[b],
          gsems[b],
      ).wait()

    def fire_scatter(k, b):
      j = k >> 2
      bi = wid * bi_per_w + (k & 3)
      for bd in range(D_MODEL // 8):
        pltpu.make_async_copy(
            tbuf_v.at[b].at[pl.ds(bd * 8, 8), pl.ds(0, BLK)],
            out_hbm.at[j, bd, bi],
            osems[b],
        ).start()

    def wait_scatter(b):
      for bd in range(D_MODEL // 8):
        pltpu.make_async_copy(
            tbuf_v.at[b].at[pl.ds(bd * 8, 8), pl.ds(0, BLK)],
            out_hbm.at[0, bd, 0],
            osems[b],
        ).wait()

    fire_gather(0, 0)

    def chunk(k, b, nb):
      wait_gather(b)

      @pl.when(k + 1 < n_chunks)
      def _():
        @pl.when(k >= NBUF - 1)
        def _():
          wait_scatter(nb)
        fire_gather(k + 1, nb)

      @plsc.parallel_loop(0, BLK, unroll=4)
      def rbody(i):
        coli = jnp.full((LANES,), i, jnp.int32)
        for db in range(D_MODEL // LANES):
          vec = rows_v[b, i, pl.ds(db * LANES, LANES)]
          plsc.store_scatter(tbuf_v.at[b], [dcols[db], coli], vec * SCALE)

      fire_scatter(k, b)

    def outer(g0, carry):
      for b in range(NBUF):
        chunk(g0 * NBUF + b, b, (b + 1) % NBUF)
      return carry

    lax.fori_loop(0, n_chunks // NBUF, outer, 0)
    for b in range(NBUF):
      wait_scatter(b)

  return kern


def kernel(x, table):
  B, S = x.shape
  idxT = x.T.reshape(-1).astype(jnp.int32)
  out5 = _make_kernel(B, S)(idxT, table)
  return out5.transpose(2, 4, 0, 1, 3).reshape(B, S, D_MODEL)


# speedup vs baseline: 1.7726x; 1.1116x over previous
import functools
import math

import jax
import jax.numpy as jnp
from jax import lax
from jax.experimental import pallas as pl
from jax.experimental.pallas import tpu as pltpu
from jax.experimental.pallas import tpu_sc as plsc

D_MODEL = 64
SCALE = math.sqrt(D_MODEL)

NC = 2
NS = 16
NW = NC * NS
LANES = 16

BLK = 128
NBUF = 4


def _make_kernel(B: int, S: int):
  bi_blocks = B // BLK
  bi_per_w = bi_blocks // NW
  n_chunks = S * bi_per_w
  idx_per_w = S * bi_per_w * BLK
  mesh = plsc.VectorSubcoreMesh(core_axis_name="c", subcore_axis_name="s")

  @functools.partial(
      pl.kernel,
      mesh=mesh,
      out_type=jax.ShapeDtypeStruct(
          (S, D_MODEL // 8, bi_blocks, 8, BLK), jnp.float32),
      scratch_types=[
          pltpu.VMEM((idx_per_w,), jnp.int32),
          pltpu.VMEM((NBUF, BLK, D_MODEL), jnp.float32),
          pltpu.VMEM((NBUF, D_MODEL, BLK + 1), jnp.float32),
          [pltpu.SemaphoreType.DMA] * NBUF,
          [pltpu.SemaphoreType.DMA] * NBUF,
          pltpu.SemaphoreType.DMA,
      ],
      compiler_params=pltpu.CompilerParams(use_tc_tiling_on_sc=False, needs_layout_passes=False),
  )
  def kern(idx_hbm, table_hbm, out_hbm, idx_v, rows_v, tbuf_v,
           gsems, osems, isem):
    wid = lax.axis_index("s") * NC + lax.axis_index("c")

    icopies = []
    for j in range(S):
      cp = pltpu.make_async_copy(
          idx_hbm.at[pl.ds(j * B + wid * (bi_per_w * BLK), bi_per_w * BLK)],
          idx_v.at[pl.ds(j * (bi_per_w * BLK), bi_per_w * BLK)],
          isem,
      )
      cp.start()
      icopies.append(cp)
    for cp in icopies:
      cp.wait()

    lane = lax.iota(jnp.int32, LANES)
    perms = [(lane + c) & (LANES - 1) for c in range(LANES)]
    dcols = [lane + db * LANES for db in range(D_MODEL // LANES)]

    def fire_gather(k, b):
      pltpu.make_async_copy(
          table_hbm.at[idx_v.at[pl.ds(k * BLK, BLK)]],
          rows_v.at[b],
          gsems[b],
      ).start()

    def wait_gather(b):
      pltpu.make_async_copy(
          table_hbm.at[idx_v.at[pl.ds(0, BLK)]],
          rows_v.at[b],
          gsems[b],
      ).wait()

    def fire_scatter(k, b):
      j = k >> 2
      bi = wid * bi_per_w + (k & 3)
      for bd in range(D_MODEL // 8):
        pltpu.make_async_copy(
            tbuf_v.at[b].at[pl.ds(bd * 8, 8), pl.ds(0, BLK)],
            out_hbm.at[j, bd, bi],
            osems[b],
        ).start()

    def wait_scatter(b):
      for bd in range(D_MODEL // 8):
        pltpu.make_async_copy(
            tbuf_v.at[b].at[pl.ds(bd * 8, 8), pl.ds(0, BLK)],
            out_hbm.at[0, bd, 0],
            osems[b],
        ).wait()

    fire_gather(0, 0)
    fire_gather(1, 1)

    def chunk(k, b, nb):
      wait_gather(b)

      @pl.when(k + 2 < n_chunks)
      def _():
        @pl.when(k >= 2)
        def _():
          wait_scatter(nb)
        fire_gather(k + 2, nb)

      @plsc.parallel_loop(0, BLK)
      def rbody(i):
        coli = jnp.full((LANES,), i, jnp.int32)
        for db in range(D_MODEL // LANES):
          vec = rows_v[b, i, pl.ds(db * LANES, LANES)]
          plsc.store_scatter(tbuf_v.at[b], [dcols[db], coli], vec * SCALE)

      fire_scatter(k, b)

    def outer(g0, carry):
      for b in range(NBUF):
        chunk(g0 * NBUF + b, b, (b + 2) % NBUF)
      return carry

    lax.fori_loop(0, n_chunks // NBUF, outer, 0)
    for b in range(NBUF):
      wait_scatter(b)

  return kern


def kernel(x, table):
  B, S = x.shape
  idxT = x.T.reshape(-1).astype(jnp.int32)
  out5 = _make_kernel(B, S)(idxT, table)
  return out5.transpose(2, 4, 0, 1, 3).reshape(B, S, D_MODEL)


# speedup vs baseline: 1.7931x; 1.0115x over previous
import functools
import math

import jax
import jax.numpy as jnp
from jax import lax
from jax.experimental import pallas as pl
from jax.experimental.pallas import tpu as pltpu
from jax.experimental.pallas import tpu_sc as plsc

D_MODEL = 64
SCALE = math.sqrt(D_MODEL)

NC = 2
NS = 16
NW = NC * NS
LANES = 16

BLK = 128
NBUF = 5


def _make_kernel(B: int, S: int):
  bi_blocks = B // BLK
  bi_per_w = bi_blocks // NW
  n_chunks = S * bi_per_w
  idx_per_w = S * bi_per_w * BLK
  mesh = plsc.VectorSubcoreMesh(core_axis_name="c", subcore_axis_name="s")

  @functools.partial(
      pl.kernel,
      mesh=mesh,
      out_type=jax.ShapeDtypeStruct(
          (S, D_MODEL // 8, bi_blocks, 8, BLK), jnp.float32),
      scratch_types=[
          pltpu.VMEM((idx_per_w,), jnp.int32),
          pltpu.VMEM((NBUF, BLK, D_MODEL), jnp.float32),
          pltpu.VMEM((NBUF, D_MODEL, BLK + 1), jnp.float32),
          [pltpu.SemaphoreType.DMA] * NBUF,
          [pltpu.SemaphoreType.DMA] * NBUF,
          pltpu.SemaphoreType.DMA,
      ],
      compiler_params=pltpu.CompilerParams(use_tc_tiling_on_sc=False, needs_layout_passes=False),
  )
  def kern(idx_hbm, table_hbm, out_hbm, idx_v, rows_v, tbuf_v,
           gsems, osems, isem):
    wid = lax.axis_index("s") * NC + lax.axis_index("c")

    icopies = []
    for j in range(S):
      cp = pltpu.make_async_copy(
          idx_hbm.at[pl.ds(j * B + wid * (bi_per_w * BLK), bi_per_w * BLK)],
          idx_v.at[pl.ds(j * (bi_per_w * BLK), bi_per_w * BLK)],
          isem,
      )
      cp.start()
      icopies.append(cp)
    for cp in icopies:
      cp.wait()

    lane = lax.iota(jnp.int32, LANES)
    perms = [(lane + c) & (LANES - 1) for c in range(LANES)]
    dcols = [lane + db * LANES for db in range(D_MODEL // LANES)]

    def fire_gather(k, b):
      pltpu.make_async_copy(
          table_hbm.at[idx_v.at[pl.ds(k * BLK, BLK)]],
          rows_v.at[b],
          gsems[b],
      ).start()

    def wait_gather(b):
      pltpu.make_async_copy(
          table_hbm.at[idx_v.at[pl.ds(0, BLK)]],
          rows_v.at[b],
          gsems[b],
      ).wait()

    def fire_scatter(k, b):
      j = k >> 2
      bi = wid * bi_per_w + (k & 3)
      for bd in range(D_MODEL // 8):
        pltpu.make_async_copy(
            tbuf_v.at[b].at[pl.ds(bd * 8, 8), pl.ds(0, BLK)],
            out_hbm.at[j, bd, bi],
            osems[b],
        ).start()

    def wait_scatter(b):
      for bd in range(D_MODEL // 8):
        pltpu.make_async_copy(
            tbuf_v.at[b].at[pl.ds(bd * 8, 8), pl.ds(0, BLK)],
            out_hbm.at[0, bd, 0],
            osems[b],
        ).wait()

    fire_gather(0, 0)
    fire_gather(1, 1)
    fire_gather(2, 2)

    def chunk(k, b, nb):
      wait_gather(b)

      @pl.when(k + 3 < n_chunks)
      def _():
        @pl.when(k >= 2)
        def _():
          wait_scatter(nb)
        fire_gather(k + 3, nb)

      @plsc.parallel_loop(0, BLK)
      def rbody(i):
        coli = jnp.full((LANES,), i, jnp.int32)
        for db in range(D_MODEL // LANES):
          vec = rows_v[b, i, pl.ds(db * LANES, LANES)]
          plsc.store_scatter(tbuf_v.at[b], [dcols[db], coli], vec * SCALE)

      fire_scatter(k, b)

    def outer(g0, carry):
      for b in range(NBUF):
        chunk(g0 * NBUF + b, b, (b + 3) % NBUF)
      return carry

    lax.fori_loop(0, n_chunks // NBUF, outer, 0)
    for b in range(NBUF):
      wait_scatter(b)

  return kern


def kernel(x, table):
  B, S = x.shape
  idxT = x.T.reshape(-1).astype(jnp.int32)
  out5 = _make_kernel(B, S)(idxT, table)
  return out5.transpose(2, 4, 0, 1, 3).reshape(B, S, D_MODEL)
